# Initial kernel scaffold; baseline (speedup 1.0000x reference)
import jax
import jax.numpy as jnp
from jax.experimental import pallas as pl


def kernel(pos, params):
    raise NotImplementedError("write your pallas kernel here")


# speedup vs baseline: 2.0336x; 2.0336x over previous
import functools

import jax
import jax.numpy as jnp
import numpy as np
from jax import lax
from jax.experimental import pallas as pl
from jax.experimental.pallas import tpu as pltpu
from jax.experimental.pallas import tpu_sc as plsc

B, N, C, DEPTH, K, H = 2, 4096, 128, 2, 16, 2
OUT_C = 128
BN = B * N
M = BN * K
P3 = 16
PG = 128


@functools.partial(jax.jit, static_argnames=("d",))
def _sc_gather(table, idx, d):
  nc = 2
  ns = 16
  nw = nc * ns
  m_per_w = M // nw
  ch = 128
  n_ch = m_per_w // ch

  mesh = plsc.VectorSubcoreMesh(core_axis_name="c", subcore_axis_name="s")

  @functools.partial(
      pl.kernel,
      out_type=jax.ShapeDtypeStruct((M, d), jnp.float32),
      mesh=mesh,
      scratch_types=[
          pltpu.VMEM((ch,), jnp.int32),
          pltpu.VMEM((ch, d), jnp.float32),
          pltpu.VMEM((ch,), jnp.int32),
          pltpu.VMEM((ch, d), jnp.float32),
          pltpu.SemaphoreType.DMA,
          pltpu.SemaphoreType.DMA,
      ],
  )
  def gather_k(table_hbm, idx_hbm, out_hbm, idx_a, rows_a, idx_b, rows_b,
               sem_a, sem_b):
    wid = lax.axis_index("s") * nc + lax.axis_index("c")
    base = wid * m_per_w

    def chunk(i, idx_v, rows_v, sem):
      off = base + i * ch
      pltpu.sync_copy(idx_hbm.at[pl.ds(off, ch)], idx_v)
      pltpu.async_copy(table_hbm.at[idx_v], rows_v, sem).wait()
      pltpu.sync_copy(rows_v, out_hbm.at[pl.ds(off, ch)])

    def body(i, _):
      chunk(2 * i, idx_a, rows_a, sem_a)
      chunk(2 * i + 1, idx_b, rows_b, sem_b)
      return 0

    lax.fori_loop(0, n_ch // 2, body, 0)

  return gather_k(table, idx)


_LT = 512


def _linear_tc(x, w, b):
  rows, cin = x.shape
  cout = w.shape[1]

  def body(x_ref, w_ref, b_ref, o_ref):
    o_ref[:] = jnp.dot(x_ref[:], w_ref[:],
                       preferred_element_type=jnp.float32) + b_ref[:]

  return pl.pallas_call(
      body,
      grid=(rows // _LT,),
      in_specs=[
          pl.BlockSpec((_LT, cin), lambda i: (i, 0)),
          pl.BlockSpec((cin, cout), lambda i: (0, 0)),
          pl.BlockSpec((1, cout), lambda i: (0, 0)),
      ],
      out_specs=pl.BlockSpec((_LT, cout), lambda i: (i, 0)),
      out_shape=jax.ShapeDtypeStruct((rows, cout), jnp.float32),
  )(x, w, b)


_T = 128


def _ln(h, g, b):
  m = jnp.mean(h, axis=-1, keepdims=True)
  v = jnp.mean((h - m) * (h - m), axis=-1, keepdims=True)
  return (h - m) * lax.rsqrt(v + 1e-5) * g + b


def _layer_body(x_ref, pos_ref, kvnb_ref, posnb_ref,
                wq_ref, bq_ref, wp1_ref, bp1_ref, wp2_ref, bp2_ref,
                wa1_ref, ba1_ref, wa2_ref, ba2_ref, wo_ref, bo_ref,
                wf1_ref, bf1_ref, g1_ref, be1_ref,
                wf2_ref, bf2_ref, g2_ref, be2_ref,
                o_ref):
  f32 = jnp.float32
  x = x_ref[:]
  q = jnp.dot(x, wq_ref[:], preferred_element_type=f32) + bq_ref[:]
  kv = kvnb_ref[:]
  k = kv[:, :C]
  v = kv[:, C:]

  pnb = posnb_ref[:, :P3]
  pd = pos_ref[:].reshape(_T, 1, P3) - pnb.reshape(_T, K, P3)
  pd = pd.reshape(_T * K, P3)
  h1 = jnp.maximum(
      jnp.dot(pd, wp1_ref[:], preferred_element_type=f32) + bp1_ref[:], 0.0)
  pe = jnp.dot(h1, wp2_ref[:], preferred_element_type=f32) + bp2_ref[:]

  qb = jnp.broadcast_to(q.reshape(_T, 1, C), (_T, K, C)).reshape(_T * K, C)
  rel = k - qb + pe
  a1 = jnp.maximum(
      jnp.dot(rel, wa1_ref[:], preferred_element_type=f32) + ba1_ref[:], 0.0)
  a = jnp.dot(a1, wa2_ref[:], preferred_element_type=f32) + ba2_ref[:]

  a3 = a.reshape(_T, K, C)
  mx = jnp.max(a3, axis=1, keepdims=True)
  e = jnp.exp(a3 - mx)
  s = jnp.sum(e, axis=1, keepdims=True)
  w = e / s
  agg = jnp.sum(w * (v.reshape(_T, K, C) + pe.reshape(_T, K, C)), axis=1)

  y = x + jnp.dot(agg, wo_ref[:], preferred_element_type=f32) + bo_ref[:]

  h = jnp.dot(y, wf1_ref[:], preferred_element_type=f32) + bf1_ref[:]
  h = _ln(h, g1_ref[:], be1_ref[:])
  h = 0.5 * h * (1.0 + lax.erf(h * np.float32(1.0 / np.sqrt(2.0))))
  h2 = jnp.dot(h, wf2_ref[:], preferred_element_type=f32) + bf2_ref[:]
  h2 = _ln(h2, g2_ref[:], be2_ref[:])
  o_ref[:] = y + h2


def _layer_tc(x, pos_pad, kvnb, posnb, wts):
  full = lambda r, c: pl.BlockSpec((r, c), lambda i: (0, 0))
  in_specs = [
      pl.BlockSpec((_T, C), lambda i: (i, 0)),
      pl.BlockSpec((_T, P3), lambda i: (i, 0)),
      pl.BlockSpec((_T * K, 2 * C), lambda i: (i, 0)),
      pl.BlockSpec((_T * K, PG), lambda i: (i, 0)),
  ]
  for a in wts:
    in_specs.append(full(a.shape[0], a.shape[1]))
  return pl.pallas_call(
      _layer_body,
      grid=(BN // _T,),
      in_specs=in_specs,
      out_specs=pl.BlockSpec((_T, C), lambda i: (i, 0)),
      out_shape=jax.ShapeDtypeStruct((BN, C), jnp.float32),
  )(x, pos_pad, kvnb, posnb, *wts)


def _knn_idx(pos):
  sq = jnp.sum(pos * pos, axis=-1)
  d2 = (sq[:, :, None] - 2.0 * jnp.einsum("bnd,bmd->bnm", pos, pos)
        + sq[:, None, :])
  _, idx = lax.top_k(-d2, K)
  return idx


def _pad_cols(a, cols):
  return jnp.pad(a, ((0, 0), (0, cols - a.shape[1])))


def _pad_rows(a, rows):
  return jnp.pad(a, ((0, rows - a.shape[0]), (0, 0)))


def kernel(pos, params):
  idx = _knn_idx(pos)
  idxg = (idx + (jnp.arange(B, dtype=idx.dtype) * N)[:, None, None])
  idxg = idxg.reshape(M).astype(jnp.int32)

  pos_flat = pos.reshape(BN, 3)
  pos_pad = _pad_cols(pos_flat, P3)
  posnb = _sc_gather(_pad_cols(pos_flat, PG), idxg, PG)

  pin = params["proj_in"]
  x = _linear_tc(pos_pad, _pad_rows(pin["w"], P3), pin["b"].reshape(1, C))

  for lp in params["layers"]:
    kv = _linear_tc(x, lp["fc_kv"]["w"], lp["fc_kv"]["b"].reshape(1, 2 * C))
    kvnb = _sc_gather(kv, idxg, 2 * C)
    wts = (
        lp["fc_q"]["w"], lp["fc_q"]["b"].reshape(1, C),
        _pad_rows(lp["pos_mlp1"]["w"], P3), lp["pos_mlp1"]["b"].reshape(1, C),
        lp["pos_mlp2"]["w"], lp["pos_mlp2"]["b"].reshape(1, C),
        lp["attn_mlp1"]["w"], lp["attn_mlp1"]["b"].reshape(1, C),
        lp["attn_mlp2"]["w"], lp["attn_mlp2"]["b"].reshape(1, C),
        lp["fc_out"]["w"], lp["fc_out"]["b"].reshape(1, C),
        lp["ffn1"]["w"], lp["ffn1"]["b"].reshape(1, 4 * C),
        lp["ln1_g"].reshape(1, 4 * C), lp["ln1_b"].reshape(1, 4 * C),
        lp["ffn2"]["w"], lp["ffn2"]["b"].reshape(1, C),
        lp["ln2_g"].reshape(1, C), lp["ln2_b"].reshape(1, C),
    )
    x = _layer_tc(x, pos_pad, kvnb, posnb, wts)

  pout = params["proj_out"]
  out = _linear_tc(x, pout["w"], pout["b"].reshape(1, OUT_C))
  return out.reshape(B, N, OUT_C)
